# Initial kernel scaffold; baseline (speedup 1.0000x reference)
import jax
import jax.numpy as jnp
from jax.experimental import pallas as pl


def kernel(cF, sF, embed, projection, pos_table, W, b_lin):
    raise NotImplementedError("write your pallas kernel here")


# speedup vs baseline: 1.7308x; 1.7308x over previous
import jax
import jax.numpy as jnp
from jax.experimental import pallas as pl


def _vq_kernel(cf_ref, sf_ref, w_ref, post_ref, bl_ref, bk_ref, emb_ref,
               embt_ref, qout_ref, ind_ref, part_ref):
    eps = 1e-5
    c = cf_ref[0]
    s = sf_ref[0]

    cm = jnp.mean(c, axis=1, keepdims=True)
    cv = jnp.mean((c - cm) ** 2, axis=1, keepdims=True)
    cs = jnp.sqrt(cv + eps)
    sm = jnp.mean(s, axis=1, keepdims=True)
    sv = jnp.mean((s - sm) ** 2, axis=1, keepdims=True)
    ss = jnp.sqrt(sv + eps)
    qn = (c - cm) / cs
    target = ss * qn + sm

    q2 = jnp.maximum(
        jnp.dot(qn, bk_ref[:], preferred_element_type=jnp.float32), 0.0) + 1e-3

    u = jnp.dot(w_ref[:], q2 + post_ref[:],
                preferred_element_type=jnp.float32) + bl_ref[:]

    emb = emb_ref[:]
    embt = embt_ref[:]
    esq = jnp.sum(emb * emb, axis=0, keepdims=True)
    lane_iota = jax.lax.broadcasted_iota(jnp.int32, (512, 1024), 1)

    gathered_cols = []
    ind_cols = []
    for h in range(16):
        x = u[:, h * 16:(h + 1) * 16]
        rowsq = jnp.sum(x * x, axis=1, keepdims=True)
        sc = jnp.dot(x, emb, preferred_element_type=jnp.float32)
        dist = rowsq - 2.0 * sc + esq
        mn = jnp.min(dist, axis=1, keepdims=True)
        idx = jnp.min(jnp.where(dist == mn, lane_iota, 1024), axis=1,
                      keepdims=True)
        oh = (lane_iota == idx).astype(jnp.float32)
        g = jnp.dot(oh, embt, preferred_element_type=jnp.float32)
        gathered_cols.append(g)
        ind_cols.append(idx)

    gathered = jnp.concatenate(gathered_cols, axis=1)
    ind = jnp.concatenate(ind_cols, axis=1)

    ind_ref[0] = ind
    qout_ref[0] = target + (gathered - target)

    diff = gathered - target
    s1 = jnp.sum(diff * diff)
    qm = jnp.mean(gathered, axis=1, keepdims=True)
    qv = jnp.mean((gathered - qm) ** 2, axis=1, keepdims=True)
    qs = jnp.sqrt(qv + eps)
    tm = jnp.mean(target, axis=1, keepdims=True)
    tv = jnp.mean((target - tm) ** 2, axis=1, keepdims=True)
    ts = jnp.sqrt(tv + eps)
    s2 = jnp.sum((qm - tm) ** 2)
    s3 = jnp.sum((qs - ts) ** 2)

    li = jax.lax.broadcasted_iota(jnp.int32, (1, 128), 1)
    part_ref[0] = jnp.where(li == 0, s1,
                            jnp.where(li == 1, s2,
                                      jnp.where(li == 2, s3, 0.0)))


def kernel(cF, sF, embed, projection, pos_table, W, b_lin):
    b, C, H, Wd = cF.shape
    dim, n_embed = embed.shape
    hw = H * Wd

    cf2 = cF.reshape(b, C, hw)
    sf2 = sF.reshape(b, C, hw)
    dn = float(dim) ** -0.25
    bk = jnp.kron(jnp.eye(H, dtype=cF.dtype), dn * projection.T)
    post = pos_table.T
    bl2 = b_lin[:, None]
    embt = embed.T

    qout, ind, part = pl.pallas_call(
        _vq_kernel,
        grid=(b,),
        in_specs=[
            pl.BlockSpec((1, C, hw), lambda i: (i, 0, 0)),
            pl.BlockSpec((1, C, hw), lambda i: (i, 0, 0)),
            pl.BlockSpec((C, C), lambda i: (0, 0)),
            pl.BlockSpec((C, hw), lambda i: (0, 0)),
            pl.BlockSpec((C, 1), lambda i: (0, 0)),
            pl.BlockSpec((hw, hw), lambda i: (0, 0)),
            pl.BlockSpec((dim, n_embed), lambda i: (0, 0)),
            pl.BlockSpec((n_embed, dim), lambda i: (0, 0)),
        ],
        out_specs=[
            pl.BlockSpec((1, C, hw), lambda i: (i, 0, 0)),
            pl.BlockSpec((1, C, H), lambda i: (i, 0, 0)),
            pl.BlockSpec((1, 1, 128), lambda i: (i, 0, 0)),
        ],
        out_shape=[
            jax.ShapeDtypeStruct((b, C, hw), jnp.float32),
            jax.ShapeDtypeStruct((b, C, H), jnp.int32),
            jax.ShapeDtypeStruct((b, 1, 128), jnp.float32),
        ],
    )(cf2, sf2, W, post, bl2, bk, embed, embt)

    quantize = qout.reshape(b, C, H, Wd)
    loss = (jnp.sum(part[:, 0, 0]) / (b * C * H * Wd)
            + 5.0 * (jnp.sum(part[:, 0, 1]) / (b * C)
                     + jnp.sum(part[:, 0, 2]) / (b * C)))
    return (quantize, ind, loss)


# speedup vs baseline: 2.6199x; 1.5137x over previous
import jax
import jax.numpy as jnp
from jax.experimental import pallas as pl


def _vq_kernel(cf_ref, sf_ref, w_ref, post_ref, bl_ref, bk_ref, embm2_ref,
               embt_ref, esq_ref, qout_ref, ind_ref, part_ref):
    eps = 1e-5
    c = cf_ref[0]
    s = sf_ref[0]

    cm = jnp.mean(c, axis=1, keepdims=True)
    cv = jnp.mean((c - cm) ** 2, axis=1, keepdims=True)
    cs = jnp.sqrt(cv + eps)
    sm = jnp.mean(s, axis=1, keepdims=True)
    sv = jnp.mean((s - sm) ** 2, axis=1, keepdims=True)
    ss = jnp.sqrt(sv + eps)
    qn = (c - cm) / cs
    target = ss * qn + sm

    q2 = jnp.maximum(
        jnp.dot(qn, bk_ref[:], preferred_element_type=jnp.float32), 0.0) + 1e-3

    u = jnp.dot(w_ref[:], q2 + post_ref[:],
                preferred_element_type=jnp.float32) + bl_ref[:]

    embm2 = embm2_ref[:]
    embt = embt_ref[:]
    esq = esq_ref[:]
    lane_iota = jax.lax.broadcasted_iota(jnp.int32, (512, 1024), 1)

    gathered_cols = []
    ind_cols = []
    for h in range(16):
        x = u[:, h * 16:(h + 1) * 16]
        dist = jnp.dot(x, embm2, preferred_element_type=jnp.float32) + esq
        idx = jnp.argmin(dist, axis=1, keepdims=True)
        oh = (lane_iota == idx).astype(jnp.float32)
        g = jnp.dot(oh, embt, preferred_element_type=jnp.float32)
        gathered_cols.append(g)
        ind_cols.append(idx.astype(jnp.int32))

    gathered = jnp.concatenate(gathered_cols, axis=1)
    ind = jnp.concatenate(ind_cols, axis=1)

    ind_ref[0] = ind
    qout_ref[0] = target + (gathered - target)

    diff = gathered - target
    s1 = jnp.sum(diff * diff)
    qm = jnp.mean(gathered, axis=1, keepdims=True)
    qv = jnp.mean((gathered - qm) ** 2, axis=1, keepdims=True)
    qs = jnp.sqrt(qv + eps)
    tm = jnp.mean(target, axis=1, keepdims=True)
    tv = jnp.mean((target - tm) ** 2, axis=1, keepdims=True)
    ts = jnp.sqrt(tv + eps)
    s2 = jnp.sum((qm - tm) ** 2)
    s3 = jnp.sum((qs - ts) ** 2)

    li = jax.lax.broadcasted_iota(jnp.int32, (1, 128), 1)
    part_ref[0] = jnp.where(li == 0, s1,
                            jnp.where(li == 1, s2,
                                      jnp.where(li == 2, s3, 0.0)))


def kernel(cF, sF, embed, projection, pos_table, W, b_lin):
    b, C, H, Wd = cF.shape
    dim, n_embed = embed.shape
    hw = H * Wd

    cf2 = cF.reshape(b, C, hw)
    sf2 = sF.reshape(b, C, hw)
    dn = float(dim) ** -0.25
    bk = jnp.kron(jnp.eye(H, dtype=cF.dtype), dn * projection.T)
    post = pos_table.T
    bl2 = b_lin[:, None]
    embt = embed.T
    embm2 = -2.0 * embed
    esq = jnp.sum(embed * embed, axis=0, keepdims=True)

    qout, ind, part = pl.pallas_call(
        _vq_kernel,
        grid=(b,),
        in_specs=[
            pl.BlockSpec((1, C, hw), lambda i: (i, 0, 0)),
            pl.BlockSpec((1, C, hw), lambda i: (i, 0, 0)),
            pl.BlockSpec((C, C), lambda i: (0, 0)),
            pl.BlockSpec((C, hw), lambda i: (0, 0)),
            pl.BlockSpec((C, 1), lambda i: (0, 0)),
            pl.BlockSpec((hw, hw), lambda i: (0, 0)),
            pl.BlockSpec((dim, n_embed), lambda i: (0, 0)),
            pl.BlockSpec((n_embed, dim), lambda i: (0, 0)),
            pl.BlockSpec((1, n_embed), lambda i: (0, 0)),
        ],
        out_specs=[
            pl.BlockSpec((1, C, hw), lambda i: (i, 0, 0)),
            pl.BlockSpec((1, C, H), lambda i: (i, 0, 0)),
            pl.BlockSpec((1, 1, 128), lambda i: (i, 0, 0)),
        ],
        out_shape=[
            jax.ShapeDtypeStruct((b, C, hw), jnp.float32),
            jax.ShapeDtypeStruct((b, C, H), jnp.int32),
            jax.ShapeDtypeStruct((b, 1, 128), jnp.float32),
        ],
    )(cf2, sf2, W, post, bl2, bk, embm2, embt, esq)

    quantize = qout.reshape(b, C, H, Wd)
    loss = (jnp.sum(part[:, 0, 0]) / (b * C * H * Wd)
            + 5.0 * (jnp.sum(part[:, 0, 1]) / (b * C)
                     + jnp.sum(part[:, 0, 2]) / (b * C)))
    return (quantize, ind, loss)
